# baseline (speedup 1.0000x reference)
import functools

import jax
import jax.numpy as jnp
from jax import lax
from jax.experimental import pallas as pl
from jax.experimental.pallas import tpu as pltpu
from jax.experimental.pallas import tpu_sc as plsc

_EMBED = 64
_B = 128


@functools.cache
def _make_lookup(seq: int, nb: int):
  info = plsc.get_sparse_core_info()
  nw = info.num_cores * info.num_subcores
  n_items = seq * nb
  items_per_w = n_items // nw
  assert n_items % nw == 0 and items_per_w % 2 == 0

  mesh = plsc.VectorSubcoreMesh(core_axis_name="c", subcore_axis_name="s")

  @functools.partial(
      pl.kernel,
      out_type=jax.ShapeDtypeStruct((seq, 8, nb, 8, _B), jnp.float32),
      mesh=mesh,
      scratch_types=[
          pltpu.VMEM((_B,), jnp.int32),
          pltpu.VMEM((_B,), jnp.int32),
          pltpu.VMEM((_B, _EMBED), jnp.float32),
          pltpu.VMEM((_B, _EMBED), jnp.float32),
          pltpu.VMEM((8, 8, _B), jnp.float32),
          pltpu.VMEM((8, 8, _B), jnp.float32),
          pltpu.SemaphoreType.DMA,
          pltpu.SemaphoreType.DMA,
          pltpu.SemaphoreType.DMA,
          pltpu.SemaphoreType.DMA,
          pltpu.SemaphoreType.DMA,
          pltpu.SemaphoreType.DMA,
      ],
      compiler_params=pltpu.CompilerParams(
          use_tc_tiling_on_sc=False, needs_layout_passes=False),
  )
  def lookup(idx_hbm, table_hbm, out_hbm,
             idx0, idx1, rows0, rows1, t0, t1,
             si0, si1, sg0, sg1, so0, so1):
    wid = lax.axis_index("s") * info.num_cores + lax.axis_index("c")
    base = wid * items_per_w
    last = base + items_per_w - 1

    iotas = [lax.iota(jnp.int32, 16) + 16 * j for j in range(8)]

    def item_sb(t):
      return t // nb, lax.rem(t, nb)

    def load_idx(t, idx_v, sem):
      s, b = item_sb(t)
      pltpu.async_copy(idx_hbm.at[s, b], idx_v, sem)

    def wait_idx(idx_v, sem):
      pltpu.make_async_copy(idx_hbm.at[0, 0], idx_v, sem).wait()

    def fire_gather(idx_v, rows_v, sem):
      pltpu.async_copy(table_hbm.at[idx_v], rows_v, sem)

    def wait_gather(rows_v, sem):
      pltpu.make_async_copy(table_hbm.at[pl.ds(0, _B)], rows_v, sem).wait()

    def transpose(rows_v, t_v):
      @plsc.parallel_loop(0, _EMBED * 8, step=1, unroll=16)
      def _body(k):
        e = k // 8
        j = lax.rem(k, 8)
        col = jnp.full((16,), e, jnp.int32)
        rows16 = iotas[0] + 16 * j
        v = plsc.load_gather(rows_v, [rows16, col])
        t_v[e // 8, lax.rem(e, 8), pl.ds(16 * j, 16)] = v

    def fire_out(t, t_v, sem):
      s, b = item_sb(t)
      for eh in range(8):
        pltpu.async_copy(t_v.at[eh], out_hbm.at[s, eh, b], sem)

    def wait_out(t_v, sem):
      pltpu.make_async_copy(t_v, out_hbm.at[0, :, 0], sem).wait()

    pltpu.sync_copy(idx_hbm.at[base // nb, lax.rem(base, nb)], idx0)
    fire_gather(idx0, rows0, sg0)
    pltpu.sync_copy(idx_hbm.at[(base + 1) // nb, lax.rem(base + 1, nb)], idx1)
    fire_gather(idx1, rows1, sg1)
    fire_out(base, t0, so0)
    fire_out(base + 1, t1, so1)

    def slot(t, idx_v, rows_v, t_v, si, sg, so):
      wait_gather(rows_v, sg)
      load_idx(jnp.minimum(t + 2, last), idx_v, si)
      wait_out(t_v, so)
      transpose(rows_v, t_v)
      fire_out(t, t_v, so)
      wait_idx(idx_v, si)
      fire_gather(idx_v, rows_v, sg)

    def body(g, carry):
      t = base + 2 * g
      slot(t, idx0, rows0, t0, si0, sg0, so0)
      slot(t + 1, idx1, rows1, t1, si1, sg1, so1)
      return carry

    lax.fori_loop(0, items_per_w // 2, body, 0)

    wait_gather(rows0, sg0)
    wait_gather(rows1, sg1)
    wait_out(t0, so0)
    wait_out(t1, so1)

  return lookup


def kernel(x, table):
  b, seq = x.shape
  nb = b // _B
  idx = x.astype(jnp.int32).T.reshape(seq, nb, _B)
  out5 = _make_lookup(seq, nb)(idx, table)
  return out5.transpose((2, 4, 0, 1, 3)).reshape(b, seq, _EMBED)


# speedup vs baseline: 1.8765x; 1.8765x over previous
import functools

import jax
import jax.numpy as jnp
from jax import lax
from jax.experimental import pallas as pl
from jax.experimental.pallas import tpu as pltpu
from jax.experimental.pallas import tpu_sc as plsc

_EMBED = 64
_B = 128


@functools.cache
def _make_lookup(seq: int, nb: int):
  info = plsc.get_sparse_core_info()
  nw = info.num_cores * info.num_subcores
  n_items = seq * nb
  items_per_w = n_items // nw
  assert n_items % nw == 0 and items_per_w % 2 == 0

  mesh = plsc.VectorSubcoreMesh(core_axis_name="c", subcore_axis_name="s")

  @functools.partial(
      pl.kernel,
      out_type=jax.ShapeDtypeStruct((seq, 8, nb, 8, _B), jnp.float32),
      mesh=mesh,
      scratch_types=[
          pltpu.VMEM((_B,), jnp.int32),
          pltpu.VMEM((_B,), jnp.int32),
          pltpu.VMEM((_B, _EMBED), jnp.float32),
          pltpu.VMEM((_B, _EMBED), jnp.float32),
          pltpu.VMEM((8, 8, _B), jnp.float32),
          pltpu.VMEM((8, 8, _B), jnp.float32),
          pltpu.SemaphoreType.DMA,
          pltpu.SemaphoreType.DMA,
          pltpu.SemaphoreType.DMA,
          pltpu.SemaphoreType.DMA,
          pltpu.SemaphoreType.DMA,
          pltpu.SemaphoreType.DMA,
      ],
      compiler_params=pltpu.CompilerParams(
          use_tc_tiling_on_sc=False, needs_layout_passes=False),
  )
  def lookup(idx_hbm, table_hbm, out_hbm,
             idx0, idx1, rows0, rows1, t0, t1,
             si0, si1, sg0, sg1, so0, so1):
    wid = lax.axis_index("s") * info.num_cores + lax.axis_index("c")
    base = wid * items_per_w
    last = base + items_per_w - 1

    iotas = [lax.iota(jnp.int32, 16) + 16 * j for j in range(8)]

    def item_sb(t):
      return t // nb, lax.rem(t, nb)

    def load_idx(t, idx_v, sem):
      s, b = item_sb(t)
      pltpu.async_copy(idx_hbm.at[s, b], idx_v, sem)

    def wait_idx(idx_v, sem):
      pltpu.make_async_copy(idx_hbm.at[0, 0], idx_v, sem).wait()

    def fire_gather(idx_v, rows_v, sem):
      pltpu.async_copy(table_hbm.at[idx_v], rows_v, sem)

    def wait_gather(rows_v, sem):
      pltpu.make_async_copy(table_hbm.at[pl.ds(0, _B)], rows_v, sem).wait()

    def transpose(rows_v, t_v):
      @plsc.parallel_loop(0, 16, step=1, unroll=4)
      def _body(d):
        tmp = iotas[0] + d
        emod = jnp.where(tmp < 16, tmp, tmp - 16)
        emodh = emod // 8
        emodl = lax.rem(emod, 8)
        for eb in range(4):
          ehvec = emodh + 2 * eb
          for j in range(8):
            bvec = iotas[j]
            evec = emod + 16 * eb
            v = plsc.load_gather(rows_v, [bvec, evec])
            plsc.store_scatter(t_v, [ehvec, emodl, bvec], v)

    def fire_out(t, t_v, sem):
      s, b = item_sb(t)
      for eh in range(8):
        pltpu.async_copy(t_v.at[eh], out_hbm.at[s, eh, b], sem)

    def wait_out(t_v, sem):
      pltpu.make_async_copy(t_v, out_hbm.at[0, :, 0], sem).wait()

    pltpu.sync_copy(idx_hbm.at[base // nb, lax.rem(base, nb)], idx0)
    fire_gather(idx0, rows0, sg0)
    pltpu.sync_copy(idx_hbm.at[(base + 1) // nb, lax.rem(base + 1, nb)], idx1)
    fire_gather(idx1, rows1, sg1)
    fire_out(base, t0, so0)
    fire_out(base + 1, t1, so1)

    def slot(t, idx_v, rows_v, t_v, si, sg, so):
      wait_gather(rows_v, sg)
      load_idx(jnp.minimum(t + 2, last), idx_v, si)
      wait_out(t_v, so)
      transpose(rows_v, t_v)
      fire_out(t, t_v, so)
      wait_idx(idx_v, si)
      fire_gather(idx_v, rows_v, sg)

    def body(g, carry):
      t = base + 2 * g
      slot(t, idx0, rows0, t0, si0, sg0, so0)
      slot(t + 1, idx1, rows1, t1, si1, sg1, so1)
      return carry

    lax.fori_loop(0, items_per_w // 2, body, 0)

    wait_gather(rows0, sg0)
    wait_gather(rows1, sg1)
    wait_out(t0, so0)
    wait_out(t1, so1)

  return lookup


def kernel(x, table):
  b, seq = x.shape
  nb = b // _B
  idx = x.astype(jnp.int32).T.reshape(seq, nb, _B)
  out5 = _make_lookup(seq, nb)(idx, table)
  return out5.transpose((2, 4, 0, 1, 3)).reshape(b, seq, _EMBED)
